# baseline (speedup 1.0000x reference)
import jax
import jax.numpy as jnp
from jax import lax
from jax.experimental import pallas as pl
from jax.experimental.pallas import tpu as pltpu
from jax.experimental.pallas import tpu_sc as plsc

L = 12
SRC = 2048 + 2
TGT = 2048 + 2
D = 1024

_TBL = L * SRC * D
_HALF = _TBL // 2
_CH = 307200
_NCH = _HALF // _CH
_TOTAL = 3 * _NCH
_NBUF = 4


def _sc_copy(enc_hbm, self_hbm, cross_hbm, out_hbm, *refs):
    bufs = refs[:_NBUF]
    rsems = refs[_NBUF:2 * _NBUF]
    wsems = refs[2 * _NBUF:]
    cid = lax.axis_index("c")
    sid = lax.axis_index("s")
    base = cid * _HALF
    srcs = (enc_hbm, self_hbm, cross_hbm)

    def rd(k):
        t, c = divmod(k, _NCH)
        b = k % _NBUF
        src = srcs[t].at[pl.ds(base + c * _CH, _CH)]
        return pltpu.make_async_copy(src, bufs[b], rsems[b])

    def wr(k):
        t, c = divmod(k, _NCH)
        b = k % _NBUF
        dst = out_hbm.at[pl.ds(t * _TBL + base + c * _CH, _CH)]
        return pltpu.make_async_copy(bufs[b], dst, wsems[b])

    @pl.when(sid == 0)
    def _():
        rd(0).start()
        for k in range(_TOTAL):
            if k + 1 < _TOTAL:
                if k + 1 >= _NBUF:
                    wr(k + 1 - _NBUF).wait()
                rd(k + 1).start()
            rd(k).wait()
            wr(k).start()
        for j in range(_TOTAL - _NBUF, _TOTAL):
            wr(j).wait()


def kernel(bsz, enc_w, self_w, cross_w):
    del bsz
    enc2 = enc_w.reshape(_TBL)
    self2 = self_w.reshape(_TBL)
    cross2 = cross_w.reshape(_TBL)
    mesh = plsc.VectorSubcoreMesh(core_axis_name="c", subcore_axis_name="s")
    run = pl.kernel(
        _sc_copy,
        out_type=jax.ShapeDtypeStruct((3 * _TBL,), jnp.float32),
        mesh=mesh,
        scratch_types=(
            [pltpu.VMEM_SHARED((_CH,), jnp.float32)] * _NBUF
            + [pltpu.SemaphoreType.DMA] * (2 * _NBUF)
        ),
    )
    out = run(enc2, self2, cross2)
    return out.reshape(3, L, SRC, D)
